# baseline (speedup 1.0000x reference)
import functools

import jax
import jax.numpy as jnp
from jax import lax
from jax.experimental import pallas as pl
from jax.experimental.pallas import tpu as pltpu
from jax.experimental.pallas import tpu_sc as plsc

N_NODES = 10000
N_EDGES = 320000
D = 128
DH = D // 2

NC = 2
NS = 16
N_TILES = NC * NS

BATCH = 128
BASE_BATCHES = 156
MAX_BATCHES = 160
MAX_EDGES_PER_TILE = MAX_BATCHES * BATCH
NSLOT = 4
N_PAD = 10240
ROWS_PER_TILE = N_PAD // NS
CHUNK = 80
N_CHUNKS = ROWS_PER_TILE // CHUNK


@functools.partial(
    pl.kernel,
    mesh=plsc.VectorSubcoreMesh(core_axis_name="c", subcore_axis_name="s"),
    compiler_params=pltpu.CompilerParams(use_tc_tiling_on_sc=False),
    out_type=jax.ShapeDtypeStruct((NC, N_PAD, DH), jnp.float32),
    scratch_types=[
        pltpu.VMEM((MAX_EDGES_PER_TILE,), jnp.int32),
        pltpu.VMEM((MAX_EDGES_PER_TILE,), jnp.int32),
        pltpu.VMEM((NSLOT, BATCH, DH), jnp.float32),
        pltpu.VMEM_SHARED((N_PAD, DH), jnp.float32),
        [pltpu.SemaphoreType.DMA] * NSLOT,
        [pltpu.SemaphoreType.DMA] * NSLOT,
    ],
)
def _aggregate(ei_hbm, feat_hbm, out_hbm,
               src_v, dst_v, rows_v, acc_sh, gsems, ssems):
    c = lax.axis_index("c")
    s = lax.axis_index("s")

    zeros16 = jnp.zeros((16,), jnp.float32)

    def zero_body(i, _):
        r = i // (DH // 16)
        col = (i % (DH // 16)) * 16
        rows_v[0, r, pl.ds(col, 16)] = zeros16
        return 0

    lax.fori_loop(0, CHUNK * (DH // 16), zero_body, 0)

    row0 = s * ROWS_PER_TILE
    zsrc = rows_v.at[0].at[pl.ds(0, CHUNK)]
    for j in range(N_CHUNKS):
        pltpu.sync_copy(zsrc, acc_sh.at[pl.ds(row0 + j * CHUNK, CHUNK)])

    ebase = s * (BASE_BATCHES * BATCH)
    last = s == NS - 1
    n_edges_here = jnp.where(last, MAX_EDGES_PER_TILE, BASE_BATCHES * BATCH)
    nb = jnp.where(last, MAX_BATCHES, BASE_BATCHES)

    @pl.when(last)
    def _():
        pltpu.sync_copy(ei_hbm.at[0, 0, pl.ds(ebase, MAX_EDGES_PER_TILE)],
                        src_v)
        pltpu.sync_copy(ei_hbm.at[1, 0, pl.ds(ebase, MAX_EDGES_PER_TILE)],
                        dst_v)

    @pl.when(jnp.logical_not(last))
    def _():
        pltpu.sync_copy(ei_hbm.at[0, 0, pl.ds(ebase, BASE_BATCHES * BATCH)],
                        src_v.at[pl.ds(0, BASE_BATCHES * BATCH)])
        pltpu.sync_copy(ei_hbm.at[1, 0, pl.ds(ebase, BASE_BATCHES * BATCH)],
                        dst_v.at[pl.ds(0, BASE_BATCHES * BATCH)])
    plsc.subcore_barrier()

    fh = feat_hbm.at[c]

    def gather_start(i, slot):
        pltpu.async_copy(fh.at[src_v.at[pl.ds(i * BATCH, BATCH)]], rows_v.at[slot], gsems[slot])

    def gather_wait(i, slot):
        pltpu.make_async_copy(fh.at[src_v.at[pl.ds(i * BATCH, BATCH)]], rows_v.at[slot],
                              gsems[slot]).wait()

    def scatter_start(i, slot):
        pltpu.async_copy(rows_v.at[slot], acc_sh.at[dst_v.at[pl.ds(i * BATCH, BATCH)]],
                         ssems[slot], add=True)

    def scatter_wait(i, slot):
        pltpu.make_async_copy(rows_v.at[slot], acc_sh.at[dst_v.at[pl.ds(i * BATCH, BATCH)]],
                              ssems[slot]).wait()

    for slot in range(NSLOT):
        gather_start(slot, slot)

    def ring_body(k, _):
        for u in range(NSLOT):
            i = NSLOT * k + u
            gather_wait(i, u)
            scatter_start(i, u)
            j = i - 2
            ju = (u + 2) % NSLOT

            @pl.when(jnp.logical_and(j >= 0, j + NSLOT < nb))
            def _():
                scatter_wait(j, ju)
                gather_start(j + NSLOT, ju)
        return 0

    lax.fori_loop(0, nb // NSLOT, ring_body, 0)
    scatter_wait(nb - 2, 2)
    scatter_wait(nb - 1, 3)
    plsc.subcore_barrier()

    stage = rows_v.at[0].at[pl.ds(0, CHUNK)]
    for j in range(N_CHUNKS):
        r = row0 + j * CHUNK
        pltpu.sync_copy(acc_sh.at[pl.ds(r, CHUNK)], stage)
        pltpu.sync_copy(stage, out_hbm.at[c, pl.ds(r, CHUNK)])


def _linear_body(h0_ref, h1_ref, w_ref, b_ref, o_ref):
    y = jnp.dot(h0_ref[0], w_ref[0], preferred_element_type=jnp.float32)
    y += jnp.dot(h1_ref[0], w_ref[1], preferred_element_type=jnp.float32)
    o_ref[...] = jnp.maximum(y + b_ref[...], 0.0)


_ROW_BLK = 1000

_linear = pl.pallas_call(
    _linear_body,
    grid=(N_NODES // _ROW_BLK,),
    in_specs=[
        pl.BlockSpec((1, _ROW_BLK, DH), lambda i: (0, i, 0)),
        pl.BlockSpec((1, _ROW_BLK, DH), lambda i: (1, i, 0)),
        pl.BlockSpec((NC, DH, D), lambda i: (0, 0, 0)),
        pl.BlockSpec((1, D), lambda i: (0, 0)),
    ],
    out_specs=pl.BlockSpec((_ROW_BLK, D), lambda i: (i, 0)),
    out_shape=jax.ShapeDtypeStruct((N_NODES, D), jnp.float32),
)


def kernel(features, edge_index, W, b):
    ei = edge_index.astype(jnp.int32).reshape(2, 1, N_EDGES)
    fh = features.reshape(N_NODES, NC, DH).swapaxes(0, 1)
    hp = _aggregate(ei, fh)
    return _linear(hp, hp, W.reshape(NC, DH, D), b.reshape(1, D))


# speedup vs baseline: 1.1631x; 1.1631x over previous
import functools

import jax
import jax.numpy as jnp
from jax import lax
from jax.experimental import pallas as pl
from jax.experimental.pallas import tpu as pltpu
from jax.experimental.pallas import tpu_sc as plsc

N_NODES = 10000
N_EDGES = 320000
D = 128

NC = 2
NS = 16
N_TILES = NC * NS

EDGES_PER_TILE = N_EDGES // N_TILES
BATCH = 80
N_BATCHES = EDGES_PER_TILE // BATCH
N_PAD = 10240
ROWS_PER_TILE = N_PAD // NS
CHUNK = 80
N_CHUNKS = ROWS_PER_TILE // CHUNK


@functools.partial(
    pl.kernel,
    mesh=plsc.VectorSubcoreMesh(core_axis_name="c", subcore_axis_name="s"),
    compiler_params=pltpu.CompilerParams(use_tc_tiling_on_sc=False),
    out_type=jax.ShapeDtypeStruct((NC, N_PAD, D), jnp.float32),
    scratch_types=[
        pltpu.VMEM((EDGES_PER_TILE,), jnp.int32),
        pltpu.VMEM((EDGES_PER_TILE,), jnp.int32),
        pltpu.VMEM((2, BATCH, D), jnp.float32),
        pltpu.VMEM_SHARED((N_PAD, D), jnp.float32),
        [pltpu.SemaphoreType.DMA] * 2,
    ],
)
def _aggregate(ei_hbm, feat_hbm, out_hbm,
               src_v, dst_v, rows_v, acc_sh, gsems):
    c = lax.axis_index("c")
    s = lax.axis_index("s")
    w = c * NS + s

    zeros16 = jnp.zeros((16,), jnp.float32)

    def zero_body(i, _):
        r = i // (D // 16)
        col = (i % (D // 16)) * 16
        rows_v[0, r, pl.ds(col, 16)] = zeros16
        return 0

    lax.fori_loop(0, CHUNK * (D // 16), zero_body, 0)

    row0 = s * ROWS_PER_TILE
    zsrc = rows_v.at[0].at[pl.ds(0, CHUNK)]
    for j in range(N_CHUNKS):
        pltpu.sync_copy(zsrc, acc_sh.at[pl.ds(row0 + j * CHUNK, CHUNK)])

    ebase = w * EDGES_PER_TILE
    pltpu.sync_copy(ei_hbm.at[0, 0, pl.ds(ebase, EDGES_PER_TILE)], src_v)
    pltpu.sync_copy(ei_hbm.at[1, 0, pl.ds(ebase, EDGES_PER_TILE)], dst_v)
    plsc.subcore_barrier()

    def gather_start(i, slot):
        idx = src_v.at[pl.ds(i * BATCH, BATCH)]
        pltpu.async_copy(feat_hbm.at[idx], rows_v.at[slot], gsems[slot])

    def gather_wait(i, slot):
        idx = src_v.at[pl.ds(i * BATCH, BATCH)]
        pltpu.make_async_copy(feat_hbm.at[idx], rows_v.at[slot],
                              gsems[slot]).wait()

    gather_start(0, 0)
    gather_start(1, 1)

    def pair_body(k, _):
        for slot in range(2):
            i = 2 * k + slot
            gather_wait(i, slot)
            didx = dst_v.at[pl.ds(i * BATCH, BATCH)]
            pltpu.sync_copy(rows_v.at[slot], acc_sh.at[didx], add=True)

            @pl.when(i + 2 < N_BATCHES)
            def _():
                gather_start(i + 2, slot)
        return 0

    lax.fori_loop(0, N_BATCHES // 2, pair_body, 0)
    i_last = N_BATCHES - 1
    gather_wait(i_last, 0)
    didx = dst_v.at[pl.ds(i_last * BATCH, BATCH)]
    pltpu.sync_copy(rows_v.at[0], acc_sh.at[didx], add=True)
    plsc.subcore_barrier()

    stage = rows_v.at[0].at[pl.ds(0, CHUNK)]
    for j in range(N_CHUNKS):
        r = row0 + j * CHUNK
        pltpu.sync_copy(acc_sh.at[pl.ds(r, CHUNK)], stage)
        pltpu.sync_copy(stage, out_hbm.at[c, pl.ds(r, CHUNK)])


def _linear_body(h0_ref, h1_ref, w_ref, b_ref, o_ref):
    h = h0_ref[0] + h1_ref[0]
    y = jnp.dot(h, w_ref[...], preferred_element_type=jnp.float32)
    o_ref[...] = jnp.maximum(y + b_ref[...], 0.0)


_ROW_BLK = 1000

_linear = pl.pallas_call(
    _linear_body,
    grid=(N_NODES // _ROW_BLK,),
    in_specs=[
        pl.BlockSpec((1, _ROW_BLK, D), lambda i: (0, i, 0)),
        pl.BlockSpec((1, _ROW_BLK, D), lambda i: (1, i, 0)),
        pl.BlockSpec((D, D), lambda i: (0, 0)),
        pl.BlockSpec((1, D), lambda i: (0, 0)),
    ],
    out_specs=pl.BlockSpec((_ROW_BLK, D), lambda i: (i, 0)),
    out_shape=jax.ShapeDtypeStruct((N_NODES, D), jnp.float32),
)


def kernel(features, edge_index, W, b):
    ei = edge_index.astype(jnp.int32).reshape(2, 1, N_EDGES)
    hp = _aggregate(ei, features)
    return _linear(hp, hp, W, b.reshape(1, D))


# speedup vs baseline: 1.2058x; 1.0367x over previous
import functools

import jax
import jax.numpy as jnp
from jax import lax
from jax.experimental import pallas as pl
from jax.experimental.pallas import tpu as pltpu
from jax.experimental.pallas import tpu_sc as plsc

N_NODES = 10000
N_EDGES = 320000
D = 128

NC = 2
NS = 16
N_TILES = NC * NS

EDGES_PER_TILE = N_EDGES // N_TILES
BATCH = 96
N_BATCHES = EDGES_PER_TILE // BATCH
TAIL = EDGES_PER_TILE - N_BATCHES * BATCH
N_PAD = 10240
ROWS_PER_TILE = N_PAD // NS
CHUNK = 80
N_CHUNKS = ROWS_PER_TILE // CHUNK


@functools.partial(
    pl.kernel,
    mesh=plsc.VectorSubcoreMesh(core_axis_name="c", subcore_axis_name="s"),
    compiler_params=pltpu.CompilerParams(use_tc_tiling_on_sc=False),
    out_type=jax.ShapeDtypeStruct((NC, N_PAD, D), jnp.float32),
    scratch_types=[
        pltpu.VMEM((EDGES_PER_TILE,), jnp.int32),
        pltpu.VMEM((EDGES_PER_TILE,), jnp.int32),
        pltpu.VMEM((2, BATCH, D), jnp.float32),
        pltpu.VMEM_SHARED((N_PAD, D), jnp.float32),
        [pltpu.SemaphoreType.DMA] * 2,
    ],
)
def _aggregate(ei_hbm, feat_hbm, out_hbm,
               src_v, dst_v, rows_v, acc_sh, gsems):
    c = lax.axis_index("c")
    s = lax.axis_index("s")
    w = c * NS + s

    zeros16 = jnp.zeros((16,), jnp.float32)

    def zero_body(i, _):
        r = i // (D // 16)
        col = (i % (D // 16)) * 16
        rows_v[0, r, pl.ds(col, 16)] = zeros16
        return 0

    lax.fori_loop(0, CHUNK * (D // 16), zero_body, 0)

    row0 = s * ROWS_PER_TILE
    zsrc = rows_v.at[0].at[pl.ds(0, CHUNK)]
    for j in range(N_CHUNKS):
        pltpu.sync_copy(zsrc, acc_sh.at[pl.ds(row0 + j * CHUNK, CHUNK)])

    ebase = w * EDGES_PER_TILE
    pltpu.sync_copy(ei_hbm.at[0, pl.ds(ebase, EDGES_PER_TILE)], src_v)
    pltpu.sync_copy(ei_hbm.at[1, pl.ds(ebase, EDGES_PER_TILE)], dst_v)
    plsc.subcore_barrier()

    def gather_start(i, slot):
        idx = src_v.at[pl.ds(i * BATCH, BATCH)]
        pltpu.async_copy(feat_hbm.at[idx], rows_v.at[slot], gsems[slot])

    def gather_wait(i, slot):
        idx = src_v.at[pl.ds(i * BATCH, BATCH)]
        pltpu.make_async_copy(feat_hbm.at[idx], rows_v.at[slot],
                              gsems[slot]).wait()

    gather_start(0, 0)
    gather_start(1, 1)

    def pair_body(k, _):
        for slot in range(2):
            i = 2 * k + slot
            gather_wait(i, slot)
            didx = dst_v.at[pl.ds(i * BATCH, BATCH)]
            pltpu.sync_copy(rows_v.at[slot], acc_sh.at[didx], add=True)

            @pl.when(i + 2 < N_BATCHES)
            def _():
                gather_start(i + 2, slot)
        return 0

    lax.fori_loop(0, N_BATCHES // 2, pair_body, 0)
    tidx = src_v.at[pl.ds(N_BATCHES * BATCH, TAIL)]
    trows = rows_v.at[0].at[pl.ds(0, TAIL)]
    pltpu.async_copy(feat_hbm.at[tidx], trows, gsems[0]).wait()
    tdidx = dst_v.at[pl.ds(N_BATCHES * BATCH, TAIL)]
    pltpu.sync_copy(trows, acc_sh.at[tdidx], add=True)
    plsc.subcore_barrier()

    stage = rows_v.at[0].at[pl.ds(0, CHUNK)]
    for j in range(N_CHUNKS):
        r = row0 + j * CHUNK
        pltpu.sync_copy(acc_sh.at[pl.ds(r, CHUNK)], stage)
        pltpu.sync_copy(stage, out_hbm.at[c, pl.ds(r, CHUNK)])


def _linear_body(h0_ref, h1_ref, w_ref, b_ref, o_ref):
    h = h0_ref[0] + h1_ref[0]
    y = jnp.dot(h, w_ref[...], preferred_element_type=jnp.float32)
    o_ref[...] = jnp.maximum(y + b_ref[...], 0.0)


_ROW_BLK = 1000

_linear = pl.pallas_call(
    _linear_body,
    grid=(N_NODES // _ROW_BLK,),
    in_specs=[
        pl.BlockSpec((1, _ROW_BLK, D), lambda i: (0, i, 0)),
        pl.BlockSpec((1, _ROW_BLK, D), lambda i: (1, i, 0)),
        pl.BlockSpec((D, D), lambda i: (0, 0)),
        pl.BlockSpec((1, D), lambda i: (0, 0)),
    ],
    out_specs=pl.BlockSpec((_ROW_BLK, D), lambda i: (i, 0)),
    out_shape=jax.ShapeDtypeStruct((N_NODES, D), jnp.float32),
)


def kernel(features, edge_index, W, b):
    ei = edge_index.astype(jnp.int32)
    hp = _aggregate(ei, features)
    return _linear(hp, hp, W, b.reshape(1, D))
